# baseline (speedup 1.0000x reference)
import jax
import jax.numpy as jnp
from jax.experimental import pallas as pl
from jax.experimental.pallas import tpu as pltpu

SEQ_BLOCK = 1024


def _add_kernel(x_ref, pe_ref, o_ref):
    o_ref[...] = x_ref[...] + pe_ref[...]


def kernel(x, pe):
    batch, seq_len, d_model = x.shape
    n_blocks = seq_len // SEQ_BLOCK
    return pl.pallas_call(
        _add_kernel,
        grid=(n_blocks, batch),
        in_specs=[
            pl.BlockSpec((1, SEQ_BLOCK, d_model), lambda i, b: (b, i, 0)),
            pl.BlockSpec((1, SEQ_BLOCK, d_model), lambda i, b: (0, i, 0)),
        ],
        out_specs=pl.BlockSpec((1, SEQ_BLOCK, d_model), lambda i, b: (b, i, 0)),
        out_shape=jax.ShapeDtypeStruct((batch, seq_len, d_model), x.dtype),
    )(x, pe[:seq_len][None])


# speedup vs baseline: 1.0754x; 1.0754x over previous
import jax
import jax.numpy as jnp
from jax.experimental import pallas as pl
from jax.experimental.pallas import tpu as pltpu

SEQ_BLOCK = 1024


def _add_kernel(x_ref, pe_ref, o_ref):
    o_ref[...] = x_ref[...] + pe_ref[...][None, :, :]


def kernel(x, pe):
    batch, seq_len, d_model = x.shape
    n_blocks = seq_len // SEQ_BLOCK
    return pl.pallas_call(
        _add_kernel,
        grid=(n_blocks,),
        in_specs=[
            pl.BlockSpec((batch, SEQ_BLOCK, d_model), lambda i: (0, i, 0)),
            pl.BlockSpec((SEQ_BLOCK, d_model), lambda i: (i, 0)),
        ],
        out_specs=pl.BlockSpec((batch, SEQ_BLOCK, d_model), lambda i: (0, i, 0)),
        out_shape=jax.ShapeDtypeStruct((batch, seq_len, d_model), x.dtype),
        compiler_params=pltpu.CompilerParams(
            dimension_semantics=("parallel",),
        ),
    )(x, pe[:seq_len])
